# baseline (speedup 1.0000x reference)
import jax
import jax.numpy as jnp
from jax import lax
from jax.experimental import pallas as pl
from jax.experimental.pallas import tpu as pltpu
from jax.experimental.pallas import tpu_sc as plsc
import functools

_NW = 32
_CH = 16
_LANES = 16


def _sc_body(x_hbm, pe_hbm, out_hbm, pe_v, x_v, sem):
    wid = lax.axis_index("s") * 2 + lax.axis_index("c")
    seq_len = pe_hbm.shape[0]
    d_model = pe_hbm.shape[1]
    batch = x_hbm.shape[0]
    span = seq_len // _NW
    base = wid * span
    nvec = (_CH * d_model) // _LANES

    def chunk_body(k, _):
        s0 = base + k * _CH
        pltpu.sync_copy(pe_hbm.at[pl.ds(s0, _CH)], pe_v)
        for b in range(batch):
            pltpu.sync_copy(x_hbm.at[b, pl.ds(s0, _CH)], x_v)

            def add_body(i, _):
                r = i // (d_model // _LANES)
                c = (i % (d_model // _LANES)) * _LANES
                x_v[r, pl.ds(c, _LANES)] = (
                    x_v[r, pl.ds(c, _LANES)] + pe_v[r, pl.ds(c, _LANES)]
                )
                return 0

            lax.fori_loop(0, nvec, add_body, 0)
            pltpu.sync_copy(x_v, out_hbm.at[b, pl.ds(s0, _CH)])
        return 0

    lax.fori_loop(0, span // _CH, chunk_body, 0)


def kernel(x, pos_embed):
    batch, seq_len, d_model = x.shape
    sc = pl.kernel(
        _sc_body,
        out_type=jax.ShapeDtypeStruct(x.shape, x.dtype),
        mesh=plsc.VectorSubcoreMesh(core_axis_name="c", subcore_axis_name="s"),
        scratch_types=[
            pltpu.VMEM((_CH, d_model), jnp.float32),
            pltpu.VMEM((_CH, d_model), jnp.float32),
            pltpu.SemaphoreType.DMA,
        ],
    )
    return sc(x, pos_embed)


# speedup vs baseline: 1.3278x; 1.3278x over previous
import jax
import jax.numpy as jnp
from jax import lax
from jax.experimental import pallas as pl
from jax.experimental.pallas import tpu as pltpu
from jax.experimental.pallas import tpu_sc as plsc

_NW = 32
_CH = 8
_LANES = 16


def _sc_body(x_hbm, pe_hbm, out_hbm, pe_v, b0, b1, b2, b3,
             sx0, sx1, sx2, sx3, so0, so1, so2, so3):
    bufs = (b0, b1, b2, b3)
    sx = (sx0, sx1, sx2, sx3)
    so = (so0, so1, so2, so3)
    wid = lax.axis_index("s") * 2 + lax.axis_index("c")
    seq_len = pe_hbm.shape[0]
    d_model = pe_hbm.shape[1]
    batch = x_hbm.shape[0]
    span = seq_len // _NW
    nchunk = span // _CH
    base = wid * span
    ncol = d_model // _LANES

    def x_load(k_s0, b, buf, sem):
        return pltpu.async_copy(x_hbm.at[b, pl.ds(k_s0, _CH)], buf, sem)

    def out_store(k_s0, b, buf, sem):
        return pltpu.async_copy(buf, out_hbm.at[b, pl.ds(k_s0, _CH)], sem)

    x_load(base, 0, bufs[0], sx[0])

    def chunk_body(k, _):
        s0 = base + k * _CH
        pltpu.sync_copy(pe_hbm.at[pl.ds(s0, _CH)], pe_v)
        for b in range(batch):
            nb = (b + 1) % 4
            if b == batch - 1:
                pltpu.make_async_copy(
                    bufs[nb], out_hbm.at[0, pl.ds(s0, _CH)], so[nb]).wait()
                pl.when(k < nchunk - 1)(
                    lambda: x_load(s0 + _CH, 0, bufs[nb], sx[nb]) and None)
            else:
                @pl.when(k > 0)
                def _():
                    pltpu.make_async_copy(
                        bufs[nb], out_hbm.at[0, pl.ds(s0, _CH)], so[nb]).wait()
                x_load(s0, b + 1, bufs[nb], sx[nb])
            pltpu.make_async_copy(
                x_hbm.at[b, pl.ds(s0, _CH)], bufs[b], sx[b]).wait()
            xv = bufs[b]

            def add_row(r, _):
                for c in range(ncol):
                    sl = pl.ds(c * _LANES, _LANES)
                    xv[r, sl] = xv[r, sl] + pe_v[r, sl]
                return 0

            lax.fori_loop(0, _CH, add_row, 0)
            out_store(s0, b, bufs[b], so[b])
        return 0

    lax.fori_loop(0, nchunk, chunk_body, 0)

    for i in range(1, 4):
        pltpu.make_async_copy(
            bufs[i], out_hbm.at[0, pl.ds(base, _CH)], so[i]).wait()


def kernel(x, pos_embed):
    batch, seq_len, d_model = x.shape
    sc = pl.kernel(
        _sc_body,
        out_type=jax.ShapeDtypeStruct(x.shape, x.dtype),
        mesh=plsc.VectorSubcoreMesh(core_axis_name="c", subcore_axis_name="s"),
        scratch_types=[pltpu.VMEM((_CH, d_model), jnp.float32)] * 5
        + [pltpu.SemaphoreType.DMA] * 8,
    )
    return sc(x, pos_embed)


# speedup vs baseline: 3.6228x; 2.7284x over previous
import jax
import jax.numpy as jnp
from jax import lax
from jax.experimental import pallas as pl
from jax.experimental.pallas import tpu as pltpu
from jax.experimental.pallas import tpu_sc as plsc

_NW = 32
_CH = 8
_LANES = 16
_S1 = 1536
_BS = 512


def _sc_body(x_hbm, pe_hbm, out_hbm, pe_v, b0, b1, b2, b3,
             sx0, sx1, sx2, sx3, so0, so1, so2, so3):
    bufs = (b0, b1, b2, b3)
    sx = (sx0, sx1, sx2, sx3)
    so = (so0, so1, so2, so3)
    wid = lax.axis_index("s") * 2 + lax.axis_index("c")
    d_model = pe_hbm.shape[1]
    batch = x_hbm.shape[0]
    span = out_hbm.shape[1] // _NW
    nchunk = span // _CH
    base = wid * span
    ncol = d_model // _LANES

    def x_load(k_s0, b, buf, sem):
        pltpu.async_copy(x_hbm.at[b, pl.ds(k_s0, _CH)], buf, sem)

    x_load(base, 0, bufs[0], sx[0])

    def chunk_body(k, _):
        s0 = base + k * _CH
        pltpu.sync_copy(pe_hbm.at[pl.ds(s0, _CH)], pe_v)
        for b in range(batch):
            nb = (b + 1) % 4
            if b == batch - 1:
                pltpu.make_async_copy(
                    bufs[nb], out_hbm.at[0, pl.ds(s0, _CH)], so[nb]).wait()

                @pl.when(k < nchunk - 1)
                def _():
                    x_load(s0 + _CH, 0, bufs[nb], sx[nb])
            else:
                @pl.when(k > 0)
                def _():
                    pltpu.make_async_copy(
                        bufs[nb], out_hbm.at[0, pl.ds(s0, _CH)], so[nb]).wait()
                x_load(s0, b + 1, bufs[nb], sx[nb])
            pltpu.make_async_copy(
                x_hbm.at[b, pl.ds(s0, _CH)], bufs[b], sx[b]).wait()
            xv = bufs[b]

            def add_row(r, _):
                for c in range(ncol):
                    sl = pl.ds(c * _LANES, _LANES)
                    xv[r, sl] = xv[r, sl] + pe_v[r, sl]
                return 0

            lax.fori_loop(0, _CH, add_row, 0)
            pltpu.async_copy(xv, out_hbm.at[b, pl.ds(s0, _CH)], so[b])
        return 0

    lax.fori_loop(0, nchunk, chunk_body, 0)

    for i in range(1, 4):
        pltpu.make_async_copy(
            bufs[i], out_hbm.at[0, pl.ds(base, _CH)], so[i]).wait()


def _tc_body(x_ref, pe_ref, o_ref):
    o_ref[...] = x_ref[...] + pe_ref[...][None, :, :]


def kernel(x, pos_embed):
    batch, seq_len, d_model = x.shape
    sc = pl.kernel(
        _sc_body,
        out_type=jax.ShapeDtypeStruct((batch, _S1, d_model), x.dtype),
        mesh=plsc.VectorSubcoreMesh(core_axis_name="c", subcore_axis_name="s"),
        scratch_types=[pltpu.VMEM((_CH, d_model), jnp.float32)] * 5
        + [pltpu.SemaphoreType.DMA] * 8,
    )
    sc_out = sc(x, pos_embed)

    off = _S1 // _BS
    grid = ((seq_len - _S1) // _BS, batch)
    tc_full = pl.pallas_call(
        _tc_body,
        grid=grid,
        in_specs=[
            pl.BlockSpec((1, _BS, d_model), lambda s, b: (b, s + off, 0)),
            pl.BlockSpec((_BS, d_model), lambda s, b: (s + off, 0)),
        ],
        out_specs=pl.BlockSpec((1, _BS, d_model), lambda s, b: (b, s + off, 0)),
        out_shape=jax.ShapeDtypeStruct(x.shape, x.dtype),
    )(x, pos_embed)

    return lax.dynamic_update_slice(tc_full, sc_out, (0, 0, 0))


# speedup vs baseline: 4.6987x; 1.2970x over previous
import jax
import jax.numpy as jnp
from jax.experimental import pallas as pl

_BS = 1024


def _add_kernel(x_ref, pe_ref, o_ref):
    o_ref[...] = x_ref[...] + pe_ref[...][None, :, :]


def kernel(x, pos_embed):
    batch, seq_len, d_model = x.shape
    grid = (seq_len // _BS, batch)
    return pl.pallas_call(
        _add_kernel,
        grid=grid,
        in_specs=[
            pl.BlockSpec((1, _BS, d_model), lambda s, b: (b, s, 0)),
            pl.BlockSpec((_BS, d_model), lambda s, b: (s, 0)),
        ],
        out_specs=pl.BlockSpec((1, _BS, d_model), lambda s, b: (b, s, 0)),
        out_shape=jax.ShapeDtypeStruct(x.shape, x.dtype),
    )(x, pos_embed)
